# baseline (speedup 1.0000x reference)
import functools

import jax
import jax.numpy as jnp
from jax import lax
from jax.experimental import pallas as pl
from jax.experimental.pallas import tpu as pltpu
from jax.experimental.pallas import tpu_sc as plsc

VOCAB = 100000
HIDDEN = 64
BATCH = 4096
SEQ = 50
LETTERS = 20
SEQ_FLAT = SEQ * LETTERS
POS_STAGE = 56
NBLK = HIDDEN // 32

CHUNKS = [(o, 128) for o in range(0, 896, 128)] + [(896, 104)]

_info = plsc.get_sparse_core_info()
NC, NS = _info.num_cores, _info.num_subcores
NW = NC * NS
B_PER_W = BATCH // NW


def _sc_body(
    ids_hbm,
    table_hbm,
    pos_hbm,
    out_hbm,
    idx_v,
    rows_v,
    pos_v,
    out_v,
    sem_g0,
    sem_g1,
    sem_i0,
    sem_i1,
    sem_o0,
    sem_o1,
):
    wid = lax.axis_index("s") * NC + lax.axis_index("c")
    pltpu.sync_copy(pos_hbm.at[pl.ds(0, POS_STAGE)], pos_v)

    sem_g = [sem_g0, sem_g1]
    sem_i = [sem_i0, sem_i1]
    sem_o = [sem_o0, sem_o1]

    def fire_idx(i, ip):
        pltpu.async_copy(ids_hbm.at[wid * B_PER_W + i], idx_v.at[ip], sem_i[ip])

    def wait_idx(ip):
        pltpu.make_async_copy(ids_hbm.at[0], idx_v.at[0], sem_i[ip]).wait()

    def fire_gathers(ip):
        for off, sz in CHUNKS:
            pltpu.async_copy(
                table_hbm.at[idx_v.at[ip, pl.ds(off, sz)]],
                rows_v.at[pl.ds(ip * SEQ_FLAT + off, sz)],
                sem_g[ip],
            )

    def wait_gathers(ip):
        pltpu.make_async_copy(
            table_hbm.at[pl.ds(0, SEQ_FLAT)],
            rows_v.at[pl.ds(0, SEQ_FLAT)],
            sem_g[ip],
        ).wait()

    def fire_out(i, ip):
        pltpu.async_copy(
            out_v.at[pl.ds(ip * SEQ, SEQ)],
            out_hbm.at[wid * B_PER_W + i],
            sem_o[ip],
        )

    def wait_out(ip):
        pltpu.make_async_copy(
            out_v.at[pl.ds(0, SEQ)],
            out_hbm.at[0],
            sem_o[ip],
        ).wait()

    def reduce_row(ip):
        row_base = ip * SEQ_FLAT
        out_base = ip * SEQ

        def per_seg(so, carry):
            for si in range(5):
                s = so * 5 + si
                r0 = row_base + s * LETTERS
                for k in range(NBLK):
                    acc_e = pos_v[s, pl.ds(k * 32, 16)]
                    acc_o = pos_v[s, pl.ds(k * 32 + 16, 16)]
                    for l in range(LETTERS):
                        v = rows_v[r0 + l, pl.ds(k * 32, 32)]
                        e, o = plsc.unpack(v, format=plsc.PackFormat.INTERLEAVED)
                        acc_e = acc_e + e
                        acc_o = acc_o + o
                    out_v[out_base + s, pl.ds(k * 32, 32)] = plsc.pack(
                        acc_e, acc_o, format=plsc.PackFormat.INTERLEAVED
                    )
            return carry

        lax.fori_loop(0, SEQ // 5, per_seg, 0)

    def row_step(i, ip):
        wait_gathers(ip)

        @pl.when(i + 2 < B_PER_W)
        def _():
            fire_idx(i + 2, ip)

        @pl.when(i + 1 < B_PER_W)
        def _():
            wait_idx(1 - ip)
            fire_gathers(1 - ip)

        @pl.when(i >= 2)
        def _():
            wait_out(ip)

        reduce_row(ip)
        fire_out(i, ip)

    fire_idx(0, 0)
    fire_idx(1, 1)
    wait_idx(0)
    fire_gathers(0)

    def step(ii, carry):
        row_step(2 * ii, 0)
        row_step(2 * ii + 1, 1)
        return carry

    lax.fori_loop(0, B_PER_W // 2, step, 0)
    wait_out(0)
    wait_out(1)


_sc_kernel = functools.partial(
    pl.kernel,
    out_type=jax.ShapeDtypeStruct((BATCH, SEQ, HIDDEN), jnp.bfloat16),
    mesh=plsc.VectorSubcoreMesh(core_axis_name="c", subcore_axis_name="s"),
    scratch_types=[
        pltpu.VMEM((2, SEQ_FLAT), jnp.int32),
        pltpu.VMEM((2 * SEQ_FLAT, HIDDEN), jnp.bfloat16),
        pltpu.VMEM((POS_STAGE, HIDDEN), jnp.float32),
        pltpu.VMEM((2 * SEQ, HIDDEN), jnp.bfloat16),
        pltpu.SemaphoreType.DMA,
        pltpu.SemaphoreType.DMA,
        pltpu.SemaphoreType.DMA,
        pltpu.SemaphoreType.DMA,
        pltpu.SemaphoreType.DMA,
        pltpu.SemaphoreType.DMA,
    ],
    compiler_params=pltpu.CompilerParams(
        use_tc_tiling_on_sc=False, needs_layout_passes=False
    ),
)(_sc_body)


@jax.jit
def kernel(input_ids, tri_table, pos_table):
    table16 = tri_table.astype(jnp.bfloat16)
    pos_de = (
        pos_table.reshape(-1, NBLK, 16, 2).transpose(0, 1, 3, 2).reshape(-1, HIDDEN)
    )
    return _sc_kernel(input_ids, table16, pos_de).astype(jnp.float32)


# speedup vs baseline: 1.1977x; 1.1977x over previous
import functools

import jax
import jax.numpy as jnp
from jax import lax
from jax.experimental import pallas as pl
from jax.experimental.pallas import tpu as pltpu
from jax.experimental.pallas import tpu_sc as plsc

VOCAB = 100000
HIDDEN = 64
BATCH = 4096
SEQ = 50
LETTERS = 20
SEQ_FLAT = SEQ * LETTERS
OUT_ROW = SEQ * HIDDEN
POS_STAGE = 56
NBLK = HIDDEN // 32

CHUNKS = [(o, 128) for o in range(0, 896, 128)] + [(896, 104)]

_info = plsc.get_sparse_core_info()
NC, NS = _info.num_cores, _info.num_subcores
NW = NC * NS
B_PER_W = BATCH // NW


def _sc_body(
    ids_hbm,
    table_hbm,
    pos_hbm,
    out_hbm,
    idx_v,
    rows_v,
    pos_v,
    out_v,
    sem_g0,
    sem_g1,
    sem_i0,
    sem_i1,
    sem_o0,
    sem_o1,
):
    wid = lax.axis_index("s") * NC + lax.axis_index("c")
    pltpu.sync_copy(pos_hbm.at[pl.ds(0, POS_STAGE)], pos_v)

    sem_g = [sem_g0, sem_g1]
    sem_i = [sem_i0, sem_i1]
    sem_o = [sem_o0, sem_o1]

    def fire_idx(i, ip):
        pltpu.async_copy(ids_hbm.at[wid * B_PER_W + i], idx_v.at[ip], sem_i[ip])

    def wait_idx(ip):
        pltpu.make_async_copy(ids_hbm.at[0], idx_v.at[0], sem_i[ip]).wait()

    def fire_gathers(ip):
        for off, sz in CHUNKS:
            pltpu.async_copy(
                table_hbm.at[idx_v.at[ip, pl.ds(off, sz)]],
                rows_v.at[pl.ds(ip * SEQ_FLAT + off, sz)],
                sem_g[ip],
            )

    def wait_gathers(ip):
        pltpu.make_async_copy(
            table_hbm.at[pl.ds(0, SEQ_FLAT)],
            rows_v.at[pl.ds(0, SEQ_FLAT)],
            sem_g[ip],
        ).wait()

    def fire_out(i, ip):
        pltpu.async_copy(
            out_v.at[pl.ds(ip * OUT_ROW, OUT_ROW)],
            out_hbm.at[wid * B_PER_W + i],
            sem_o[ip],
        )

    def wait_out(ip):
        pltpu.make_async_copy(
            out_v.at[pl.ds(0, OUT_ROW)],
            out_hbm.at[0],
            sem_o[ip],
        ).wait()

    def reduce_row(ip):
        row_base = ip * SEQ_FLAT
        out_base = ip * OUT_ROW
        ve = lax.iota(jnp.int32, 16) * 2

        def per_seg(so, carry):
            for si in range(5):
                s = so * 5 + si
                r0 = row_base + s * LETTERS
                for k in range(NBLK):
                    acc_e = pos_v[s, pl.ds(k * 32, 16)]
                    acc_o = pos_v[s, pl.ds(k * 32 + 16, 16)]
                    for l in range(LETTERS):
                        v = rows_v[r0 + l, pl.ds(k * 32, 32)]
                        e, o = plsc.unpack(v, format=plsc.PackFormat.INTERLEAVED)
                        acc_e = acc_e + e
                        acc_o = acc_o + o
                    base = out_base + s * HIDDEN + 32 * k
                    plsc.store_scatter(out_v, [ve + base], acc_e)
                    plsc.store_scatter(out_v, [ve + (base + 1)], acc_o)
            return carry

        lax.fori_loop(0, SEQ // 5, per_seg, 0)

    def row_step(i, ip):
        wait_gathers(ip)

        @pl.when(i + 2 < B_PER_W)
        def _():
            fire_idx(i + 2, ip)

        @pl.when(i + 1 < B_PER_W)
        def _():
            wait_idx(1 - ip)
            fire_gathers(1 - ip)

        @pl.when(i >= 2)
        def _():
            wait_out(ip)

        reduce_row(ip)
        fire_out(i, ip)

    fire_idx(0, 0)
    fire_idx(1, 1)
    wait_idx(0)
    fire_gathers(0)

    def step(ii, carry):
        row_step(2 * ii, 0)
        row_step(2 * ii + 1, 1)
        return carry

    lax.fori_loop(0, B_PER_W // 2, step, 0)
    wait_out(0)
    wait_out(1)


_sc_kernel = functools.partial(
    pl.kernel,
    out_type=jax.ShapeDtypeStruct((BATCH, SEQ * HIDDEN), jnp.float32),
    mesh=plsc.VectorSubcoreMesh(core_axis_name="c", subcore_axis_name="s"),
    scratch_types=[
        pltpu.VMEM((2, SEQ_FLAT), jnp.int32),
        pltpu.VMEM((2 * SEQ_FLAT, HIDDEN), jnp.bfloat16),
        pltpu.VMEM((POS_STAGE, HIDDEN), jnp.float32),
        pltpu.VMEM((2 * OUT_ROW,), jnp.float32),
        pltpu.SemaphoreType.DMA,
        pltpu.SemaphoreType.DMA,
        pltpu.SemaphoreType.DMA,
        pltpu.SemaphoreType.DMA,
        pltpu.SemaphoreType.DMA,
        pltpu.SemaphoreType.DMA,
    ],
    compiler_params=pltpu.CompilerParams(
        use_tc_tiling_on_sc=False, needs_layout_passes=False
    ),
)(_sc_body)


@jax.jit
def kernel(input_ids, tri_table, pos_table):
    table16 = tri_table.astype(jnp.bfloat16)
    pos_de = (
        pos_table.reshape(-1, NBLK, 16, 2).transpose(0, 1, 3, 2).reshape(-1, HIDDEN)
    )
    return _sc_kernel(input_ids, table16, pos_de).reshape(BATCH, SEQ, HIDDEN)


# speedup vs baseline: 1.2002x; 1.0021x over previous
import functools

import jax
import jax.numpy as jnp
from jax import lax
from jax.experimental import pallas as pl
from jax.experimental.pallas import tpu as pltpu
from jax.experimental.pallas import tpu_sc as plsc

VOCAB = 100000
HIDDEN = 64
BATCH = 4096
SEQ = 50
LETTERS = 20
SEQ_FLAT = SEQ * LETTERS
OUT_ROW = SEQ * HIDDEN
POS_STAGE = 56
NBLK = HIDDEN // 32

CHUNKS = [(0, SEQ_FLAT)]

_info = plsc.get_sparse_core_info()
NC, NS = _info.num_cores, _info.num_subcores
NW = NC * NS
B_PER_W = BATCH // NW


def _sc_body(
    ids_hbm,
    table_hbm,
    pos_hbm,
    out_hbm,
    idx_v,
    rows_v,
    pos_v,
    out_v,
    sem_g0,
    sem_g1,
    sem_i0,
    sem_i1,
    sem_o0,
    sem_o1,
):
    wid = lax.axis_index("s") * NC + lax.axis_index("c")
    pltpu.sync_copy(pos_hbm.at[pl.ds(0, POS_STAGE)], pos_v)

    sem_g = [sem_g0, sem_g1]
    sem_i = [sem_i0, sem_i1]
    sem_o = [sem_o0, sem_o1]

    def fire_idx(i, ip):
        pltpu.async_copy(ids_hbm.at[wid * B_PER_W + i], idx_v.at[ip], sem_i[ip])

    def wait_idx(ip):
        pltpu.make_async_copy(ids_hbm.at[0], idx_v.at[0], sem_i[ip]).wait()

    def fire_gathers(ip):
        for off, sz in CHUNKS:
            pltpu.async_copy(
                table_hbm.at[idx_v.at[ip, pl.ds(off, sz)]],
                rows_v.at[pl.ds(ip * SEQ_FLAT + off, sz)],
                sem_g[ip],
            )

    def wait_gathers(ip):
        pltpu.make_async_copy(
            table_hbm.at[pl.ds(0, SEQ_FLAT)],
            rows_v.at[pl.ds(0, SEQ_FLAT)],
            sem_g[ip],
        ).wait()

    def fire_out(i, ip):
        pltpu.async_copy(
            out_v.at[pl.ds(ip * OUT_ROW, OUT_ROW)],
            out_hbm.at[wid * B_PER_W + i],
            sem_o[ip],
        )

    def wait_out(ip):
        pltpu.make_async_copy(
            out_v.at[pl.ds(0, OUT_ROW)],
            out_hbm.at[0],
            sem_o[ip],
        ).wait()

    def reduce_row(ip):
        row_base = ip * SEQ_FLAT
        out_base = ip * OUT_ROW
        ve = lax.iota(jnp.int32, 16) * 2

        def per_seg(so, carry):
            for si in range(5):
                s = so * 5 + si
                r0 = row_base + s * LETTERS
                for k in range(NBLK):
                    acc_e = pos_v[s, pl.ds(k * 32, 16)]
                    acc_o = pos_v[s, pl.ds(k * 32 + 16, 16)]
                    for l in range(LETTERS):
                        v = rows_v[r0 + l, pl.ds(k * 32, 32)]
                        e, o = plsc.unpack(v, format=plsc.PackFormat.INTERLEAVED)
                        acc_e = acc_e + e
                        acc_o = acc_o + o
                    base = out_base + s * HIDDEN + 32 * k
                    plsc.store_scatter(out_v, [ve + base], acc_e)
                    plsc.store_scatter(out_v, [ve + (base + 1)], acc_o)
            return carry

        lax.fori_loop(0, SEQ // 5, per_seg, 0)

    def row_step(i, ip):
        wait_gathers(ip)

        @pl.when(i + 2 < B_PER_W)
        def _():
            fire_idx(i + 2, ip)

        @pl.when(i + 1 < B_PER_W)
        def _():
            wait_idx(1 - ip)
            fire_gathers(1 - ip)

        @pl.when(i >= 2)
        def _():
            wait_out(ip)

        reduce_row(ip)
        fire_out(i, ip)

    fire_idx(0, 0)
    fire_idx(1, 1)
    wait_idx(0)
    fire_gathers(0)

    def step(ii, carry):
        row_step(2 * ii, 0)
        row_step(2 * ii + 1, 1)
        return carry

    lax.fori_loop(0, B_PER_W // 2, step, 0)
    wait_out(0)
    wait_out(1)


_sc_kernel = functools.partial(
    pl.kernel,
    out_type=jax.ShapeDtypeStruct((BATCH, SEQ * HIDDEN), jnp.float32),
    mesh=plsc.VectorSubcoreMesh(core_axis_name="c", subcore_axis_name="s"),
    scratch_types=[
        pltpu.VMEM((2, SEQ_FLAT), jnp.int32),
        pltpu.VMEM((2 * SEQ_FLAT, HIDDEN), jnp.bfloat16),
        pltpu.VMEM((POS_STAGE, HIDDEN), jnp.float32),
        pltpu.VMEM((2 * OUT_ROW,), jnp.float32),
        pltpu.SemaphoreType.DMA,
        pltpu.SemaphoreType.DMA,
        pltpu.SemaphoreType.DMA,
        pltpu.SemaphoreType.DMA,
        pltpu.SemaphoreType.DMA,
        pltpu.SemaphoreType.DMA,
    ],
    compiler_params=pltpu.CompilerParams(
        use_tc_tiling_on_sc=False, needs_layout_passes=False
    ),
)(_sc_body)


@jax.jit
def kernel(input_ids, tri_table, pos_table):
    table16 = tri_table.astype(jnp.bfloat16)
    pos_de = (
        pos_table.reshape(-1, NBLK, 16, 2).transpose(0, 1, 3, 2).reshape(-1, HIDDEN)
    )
    return _sc_kernel(input_ids, table16, pos_de).reshape(BATCH, SEQ, HIDDEN)
